# baseline (speedup 1.0000x reference)
import functools

import jax
import jax.numpy as jnp
from jax import lax
from jax.experimental import pallas as pl
from jax.experimental.pallas import tpu as pltpu
from jax.experimental.pallas import tpu_sc as plsc

B, L, D = 4096, 50, 64
N = B * L
NT = 4
DOUT = NT * D

_info = plsc.get_sparse_core_info()
NC, NS = _info.num_cores, _info.num_subcores
NW = NC * NS
B_PER_W = B // NW
PAIRS = B_PER_W // 2


def _make_kernel():
    mesh = plsc.VectorSubcoreMesh(core_axis_name="c", subcore_axis_name="s")

    LP = 64

    def set_scratch():
        return (
            [pltpu.VMEM((LP,), jnp.int32) for _ in range(NT)]
            + [pltpu.VMEM((LP,), jnp.int32) for _ in range(NT)]
            + [pltpu.VMEM((LP,), jnp.int32) for _ in range(NT)]
            + [pltpu.VMEM((L, 2 * D), jnp.float32) for _ in range(NT)]
            + [pltpu.VMEM((L, DOUT), jnp.float32)]
        )

    @functools.partial(
        pl.kernel,
        mesh=mesh,
        out_type=jax.ShapeDtypeStruct((B, L, DOUT), jnp.float32),
        scratch_types=[
            *set_scratch(), *set_scratch(),
            pltpu.SemaphoreType.DMA, pltpu.SemaphoreType.DMA,
            pltpu.SemaphoreType.DMA, pltpu.SemaphoreType.DMA,
            pltpu.SemaphoreType.DMA, pltpu.SemaphoreType.DMA,
        ],
        compiler_params=pltpu.CompilerParams(needs_layout_passes=False),
    )
    def k(gids, sids, cids, pids, gt2, st2, ct2, pt2, out, *refs):
        wid = lax.axis_index("s") * NC + lax.axis_index("c")
        b0 = wid * B_PER_W

        ids = (gids, sids, cids, pids)
        tabs = (gt2, st2, ct2, pt2)
        per_set = 3 * NT + NT + 1
        sets = []
        for s in range(2):
            r = refs[s * per_set:(s + 1) * per_set]
            sets.append(dict(raw=r[0:NT], gix=r[NT:2 * NT],
                             par=r[2 * NT:3 * NT], bg=r[3 * NT:4 * NT],
                             cc=r[4 * NT]))
        sem_p = refs[2 * per_set:2 * per_set + 2]
        sem_g = refs[2 * per_set + 2:2 * per_set + 4]
        sem_w = refs[2 * per_set + 4:2 * per_set + 6]

        def prefetch_ids(s, batch):
            @pl.when(batch < B_PER_W)
            def _():
                for t in range(NT):
                    pltpu.async_copy(ids[t].at[pl.ds((b0 + batch) * LP, LP)],
                                     sets[s]["raw"][t], sem_p[s])

        def wait_ids(s):
            for t in range(NT):
                pltpu.make_async_copy(ids[t].at[pl.ds(b0 * LP, LP)],
                                      sets[s]["raw"][t], sem_p[s]).wait()

        def build_gix(s):
            st = sets[s]
            for t in range(NT):
                raw, gix, par = st["raw"][t], st["gix"][t], st["par"][t]
                for g in range(4):
                    v = raw[pl.ds(16 * g, 16)]
                    gix[pl.ds(16 * g, 16)] = v >> 1
                    par[pl.ds(16 * g, 16)] = v & 1

        def fire_gathers(s):
            st = sets[s]
            return [pltpu.async_copy(
                        tabs[t].at[st["gix"][t].at[pl.ds(0, L)]],
                        st["bg"][t], sem_g[s])
                    for t in range(NT)]

        iota16 = lax.broadcasted_iota(jnp.int32, (16,), 0)

        def select(s):
            st = sets[s]
            cc = st["cc"]
            pars = [[st["par"][t][pl.ds(16 * g, 16)] * D for g in range(4)]
                    for t in range(NT)]

            def group(g, _):
                rowv = iota16 + g * 16
                mask = rowv < L
                rowg = jnp.minimum(rowv, L - 1)
                for t in range(NT):
                    p = pars[t]
                    odd = (g & 1) == 1
                    lo = lax.select_n(odd, p[0], p[1])
                    hi = lax.select_n(odd, p[2], p[3])
                    src0 = lax.select_n(g >= 2, lo, hi)
                    bg = st["bg"][t]
                    for comp in range(D):
                        val = plsc.load_gather(bg, [rowg, src0 + comp])
                        plsc.store_scatter(
                            cc, [rowv, iota16 * 0 + (t * D + comp)],
                            val, mask=mask)
                return 0

            lax.fori_loop(0, 4, group, 0)

        def write(s, batch):
            pltpu.async_copy(sets[s]["cc"], out.at[b0 + batch], sem_w[s])

        def drain_write(s):
            pltpu.make_async_copy(out.at[b0], sets[s]["cc"], sem_w[s]).wait()

        prefetch_ids(0, 0)
        prefetch_ids(1, 1)

        def pair(j, _):
            c0, c1 = 2 * j, 2 * j + 1
            wait_ids(0)
            build_gix(0)
            prefetch_ids(0, c0 + 2)
            g0 = fire_gathers(0)

            wait_ids(1)
            build_gix(1)
            prefetch_ids(1, c1 + 2)
            g1 = fire_gathers(1)

            for c in g0:
                c.wait()

            @pl.when(j > 0)
            def _():
                drain_write(0)
            select(0)
            write(0, c0)
            for c in g1:
                c.wait()

            @pl.when(j > 0)
            def _():
                drain_write(1)
            select(1)
            write(1, c1)
            return 0

        lax.fori_loop(0, PAIRS, pair, 0)
        drain_write(0)
        drain_write(1)

    return k


_kern = _make_kernel()


def _pad_ids(x):
    return jnp.pad(x, ((0, 0), (0, 64 - L))).reshape(-1)


def kernel(goods_ids, shop_ids, cate_ids, goods_prices,
           goods_table, shop_table, cate_table, price_table):
    return _kern(_pad_ids(goods_ids), _pad_ids(shop_ids),
                 _pad_ids(cate_ids), _pad_ids(goods_prices),
                 goods_table.reshape(-1, 2 * D),
                 shop_table.reshape(-1, 2 * D),
                 cate_table.reshape(-1, 2 * D),
                 price_table.reshape(-1, 2 * D))


# speedup vs baseline: 1.0008x; 1.0008x over previous
import functools

import jax
import jax.numpy as jnp
from jax import lax
from jax.experimental import pallas as pl
from jax.experimental.pallas import tpu as pltpu
from jax.experimental.pallas import tpu_sc as plsc

B, L, D = 4096, 50, 64
N = B * L
NT = 4
DOUT = NT * D

_info = plsc.get_sparse_core_info()
NC, NS = _info.num_cores, _info.num_subcores
NW = NC * NS
B_PER_W = B // NW
PAIRS = B_PER_W // 2


def _make_kernel():
    mesh = plsc.VectorSubcoreMesh(core_axis_name="c", subcore_axis_name="s")

    LP = 64

    def set_scratch():
        return (
            [pltpu.VMEM((LP,), jnp.int32) for _ in range(NT)]
            + [pltpu.VMEM((LP,), jnp.int32) for _ in range(NT)]
            + [pltpu.VMEM((LP,), jnp.int32) for _ in range(NT)]
            + [pltpu.VMEM((L, 2 * D), jnp.float32) for _ in range(NT)]
            + [pltpu.VMEM((L, DOUT), jnp.float32)]
        )

    @functools.partial(
        pl.kernel,
        mesh=mesh,
        out_type=jax.ShapeDtypeStruct((B, L, DOUT), jnp.float32),
        scratch_types=[
            *set_scratch(), *set_scratch(),
            pltpu.SemaphoreType.DMA, pltpu.SemaphoreType.DMA,
            pltpu.SemaphoreType.DMA, pltpu.SemaphoreType.DMA,
            pltpu.SemaphoreType.DMA, pltpu.SemaphoreType.DMA,
        ],
        compiler_params=pltpu.CompilerParams(needs_layout_passes=False,
                                             disable_bounds_checks=True),
    )
    def k(gids, sids, cids, pids, gt2, st2, ct2, pt2, out, *refs):
        wid = lax.axis_index("s") * NC + lax.axis_index("c")
        b0 = wid * B_PER_W

        ids = (gids, sids, cids, pids)
        tabs = (gt2, st2, ct2, pt2)
        per_set = 3 * NT + NT + 1
        sets = []
        for s in range(2):
            r = refs[s * per_set:(s + 1) * per_set]
            sets.append(dict(raw=r[0:NT], gix=r[NT:2 * NT],
                             par=r[2 * NT:3 * NT], bg=r[3 * NT:4 * NT],
                             cc=r[4 * NT]))
        sem_p = refs[2 * per_set:2 * per_set + 2]
        sem_g = refs[2 * per_set + 2:2 * per_set + 4]
        sem_w = refs[2 * per_set + 4:2 * per_set + 6]

        def prefetch_ids(s, batch):
            @pl.when(batch < B_PER_W)
            def _():
                for t in range(NT):
                    pltpu.async_copy(ids[t].at[pl.ds((b0 + batch) * LP, LP)],
                                     sets[s]["raw"][t], sem_p[s])

        def wait_ids(s):
            for t in range(NT):
                pltpu.make_async_copy(ids[t].at[pl.ds(b0 * LP, LP)],
                                      sets[s]["raw"][t], sem_p[s]).wait()

        def build_gix(s):
            st = sets[s]
            for t in range(NT):
                raw, gix, par = st["raw"][t], st["gix"][t], st["par"][t]
                for g in range(4):
                    v = raw[pl.ds(16 * g, 16)]
                    gix[pl.ds(16 * g, 16)] = v >> 1
                    par[pl.ds(16 * g, 16)] = v & 1

        def fire_gathers(s):
            st = sets[s]
            return [pltpu.async_copy(
                        tabs[t].at[st["gix"][t].at[pl.ds(0, L)]],
                        st["bg"][t], sem_g[s])
                    for t in range(NT)]

        iota16 = lax.broadcasted_iota(jnp.int32, (16,), 0)

        def select(s):
            st = sets[s]
            cc = st["cc"]
            pars = [[st["par"][t][pl.ds(16 * g, 16)] * D for g in range(4)]
                    for t in range(NT)]

            def group(g, _):
                rowv = iota16 + g * 16
                mask = rowv < L
                rowg = jnp.minimum(rowv, L - 1)
                for t in range(NT):
                    p = pars[t]
                    odd = (g & 1) == 1
                    lo = lax.select_n(odd, p[0], p[1])
                    hi = lax.select_n(odd, p[2], p[3])
                    src0 = lax.select_n(g >= 2, lo, hi)
                    bg = st["bg"][t]
                    for comp in range(D):
                        val = plsc.load_gather(bg, [rowg, src0 + comp])
                        plsc.store_scatter(
                            cc, [rowv, iota16 * 0 + (t * D + comp)],
                            val, mask=mask)
                return 0

            lax.fori_loop(0, 4, group, 0)

        def write(s, batch):
            pltpu.async_copy(sets[s]["cc"], out.at[b0 + batch], sem_w[s])

        def drain_write(s):
            pltpu.make_async_copy(out.at[b0], sets[s]["cc"], sem_w[s]).wait()

        prefetch_ids(0, 0)
        prefetch_ids(1, 1)

        def pair(j, _):
            c0, c1 = 2 * j, 2 * j + 1
            wait_ids(0)
            build_gix(0)
            prefetch_ids(0, c0 + 2)
            g0 = fire_gathers(0)

            wait_ids(1)
            build_gix(1)
            prefetch_ids(1, c1 + 2)
            g1 = fire_gathers(1)

            for c in g0:
                c.wait()

            @pl.when(j > 0)
            def _():
                drain_write(0)
            select(0)
            write(0, c0)
            for c in g1:
                c.wait()

            @pl.when(j > 0)
            def _():
                drain_write(1)
            select(1)
            write(1, c1)
            return 0

        lax.fori_loop(0, PAIRS, pair, 0)
        drain_write(0)
        drain_write(1)

    return k


_kern = _make_kernel()


def _pad_ids(x):
    return jnp.pad(x, ((0, 0), (0, 64 - L))).reshape(-1)


def kernel(goods_ids, shop_ids, cate_ids, goods_prices,
           goods_table, shop_table, cate_table, price_table):
    return _kern(_pad_ids(goods_ids), _pad_ids(shop_ids),
                 _pad_ids(cate_ids), _pad_ids(goods_prices),
                 goods_table.reshape(-1, 2 * D),
                 shop_table.reshape(-1, 2 * D),
                 cate_table.reshape(-1, 2 * D),
                 price_table.reshape(-1, 2 * D))


# speedup vs baseline: 1.0566x; 1.0558x over previous
import functools

import jax
import jax.numpy as jnp
from jax import lax
from jax.experimental import pallas as pl
from jax.experimental.pallas import tpu as pltpu
from jax.experimental.pallas import tpu_sc as plsc

B, L, D = 4096, 50, 64
N = B * L
NT = 4
DOUT = NT * D

_info = plsc.get_sparse_core_info()
NC, NS = _info.num_cores, _info.num_subcores
NW = NC * NS
B_PER_W = B // NW
PAIRS = B_PER_W // 2


def _make_kernel():
    mesh = plsc.VectorSubcoreMesh(core_axis_name="c", subcore_axis_name="s")

    LP = 64

    def set_scratch():
        return (
            [pltpu.VMEM((LP,), jnp.int32) for _ in range(NT)]
            + [pltpu.VMEM((LP,), jnp.int32) for _ in range(NT)]
            + [pltpu.VMEM((LP,), jnp.int32) for _ in range(NT)]
            + [pltpu.VMEM((L, 2 * D), jnp.float32) for _ in range(NT)]
            + [pltpu.VMEM((L, DOUT), jnp.float32)]
        )

    @functools.partial(
        pl.kernel,
        mesh=mesh,
        out_type=jax.ShapeDtypeStruct((B, L, DOUT), jnp.float32),
        scratch_types=[
            *set_scratch(), *set_scratch(),
            pltpu.SemaphoreType.DMA, pltpu.SemaphoreType.DMA,
            pltpu.SemaphoreType.DMA, pltpu.SemaphoreType.DMA,
            pltpu.SemaphoreType.DMA, pltpu.SemaphoreType.DMA,
        ],
        compiler_params=pltpu.CompilerParams(needs_layout_passes=False,
                                             disable_bounds_checks=True),
    )
    def k(gids, sids, cids, pids, gt2, st2, ct2, pt2, out, *refs):
        wid = lax.axis_index("s") * NC + lax.axis_index("c")
        b0 = wid * B_PER_W

        ids = (gids, sids, cids, pids)
        tabs = (gt2, st2, ct2, pt2)
        per_set = 3 * NT + NT + 1
        sets = []
        for s in range(2):
            r = refs[s * per_set:(s + 1) * per_set]
            sets.append(dict(raw=r[0:NT], gix=r[NT:2 * NT],
                             par=r[2 * NT:3 * NT], bg=r[3 * NT:4 * NT],
                             cc=r[4 * NT]))
        sem_p = refs[2 * per_set:2 * per_set + 2]
        sem_g = refs[2 * per_set + 2:2 * per_set + 4]
        sem_w = refs[2 * per_set + 4:2 * per_set + 6]

        def prefetch_ids(s, batch):
            @pl.when(batch < B_PER_W)
            def _():
                for t in range(NT):
                    pltpu.async_copy(ids[t].at[pl.ds((b0 + batch) * LP, LP)],
                                     sets[s]["raw"][t], sem_p[s])

        def wait_ids(s):
            for t in range(NT):
                pltpu.make_async_copy(ids[t].at[pl.ds(b0 * LP, LP)],
                                      sets[s]["raw"][t], sem_p[s]).wait()

        def build_gix(s):
            st = sets[s]
            for t in range(NT):
                raw, gix, par = st["raw"][t], st["gix"][t], st["par"][t]
                for g in range(4):
                    v = raw[pl.ds(16 * g, 16)]
                    gix[pl.ds(16 * g, 16)] = v >> 1
                    par[pl.ds(16 * g, 16)] = v & 1

        def fire_gathers(s):
            st = sets[s]
            return [pltpu.async_copy(
                        tabs[t].at[st["gix"][t].at[pl.ds(0, L)]],
                        st["bg"][t], sem_g[s])
                    for t in range(NT)]

        iota16 = lax.broadcasted_iota(jnp.int32, (16,), 0)

        def select(s):
            st = sets[s]
            cc = st["cc"]
            pars = [[st["par"][t][pl.ds(16 * g, 16)] * D for g in range(4)]
                    for t in range(NT)]

            def group(g, _):
                rowv = iota16 + g * 16
                mask = rowv < L
                rowg = jnp.minimum(rowv, L - 1)
                odd = (g & 1) == 1
                srcs = []
                for t in range(NT):
                    p = pars[t]
                    lo = lax.select_n(odd, p[0], p[1])
                    hi = lax.select_n(odd, p[2], p[3])
                    srcs.append(lax.select_n(g >= 2, lo, hi))
                for comp in range(D):
                    vals = [plsc.load_gather(st["bg"][t],
                                             [rowg, srcs[t] + comp])
                            for t in range(NT)]
                    for t in range(NT):
                        plsc.store_scatter(
                            cc, [rowv, iota16 * 0 + (t * D + comp)],
                            vals[t], mask=mask)
                return 0

            lax.fori_loop(0, 4, group, 0)

        def write(s, batch):
            pltpu.async_copy(sets[s]["cc"], out.at[b0 + batch], sem_w[s])

        def drain_write(s):
            pltpu.make_async_copy(out.at[b0], sets[s]["cc"], sem_w[s]).wait()

        prefetch_ids(0, 0)
        prefetch_ids(1, 1)

        def pair(j, _):
            c0, c1 = 2 * j, 2 * j + 1
            wait_ids(0)
            build_gix(0)
            prefetch_ids(0, c0 + 2)
            g0 = fire_gathers(0)

            wait_ids(1)
            build_gix(1)
            prefetch_ids(1, c1 + 2)
            g1 = fire_gathers(1)

            for c in g0:
                c.wait()

            @pl.when(j > 0)
            def _():
                drain_write(0)
            select(0)
            write(0, c0)
            for c in g1:
                c.wait()

            @pl.when(j > 0)
            def _():
                drain_write(1)
            select(1)
            write(1, c1)
            return 0

        lax.fori_loop(0, PAIRS, pair, 0)
        drain_write(0)
        drain_write(1)

    return k


_kern = _make_kernel()


def _pad_ids(x):
    return jnp.pad(x, ((0, 0), (0, 64 - L))).reshape(-1)


def kernel(goods_ids, shop_ids, cate_ids, goods_prices,
           goods_table, shop_table, cate_table, price_table):
    return _kern(_pad_ids(goods_ids), _pad_ids(shop_ids),
                 _pad_ids(cate_ids), _pad_ids(goods_prices),
                 goods_table.reshape(-1, 2 * D),
                 shop_table.reshape(-1, 2 * D),
                 cate_table.reshape(-1, 2 * D),
                 price_table.reshape(-1, 2 * D))


# speedup vs baseline: 2.6895x; 2.5454x over previous
import functools

import jax
import jax.numpy as jnp
from jax import lax
from jax.experimental import pallas as pl
from jax.experimental.pallas import tpu as pltpu
from jax.experimental.pallas import tpu_sc as plsc

B, L, D = 4096, 50, 64
N = B * L
NT = 4
DOUT = NT * D

_info = plsc.get_sparse_core_info()
NC, NS = _info.num_cores, _info.num_subcores
NW = NC * NS
B_PER_W = B // NW
PAIRS = B_PER_W // 2


def _make_kernel():
    mesh = plsc.VectorSubcoreMesh(core_axis_name="c", subcore_axis_name="s")

    LP = 64

    def set_scratch():
        return (
            [pltpu.VMEM((LP,), jnp.int32) for _ in range(NT)]
            + [pltpu.VMEM((LP,), jnp.int32) for _ in range(NT)]
            + [pltpu.VMEM((LP,), jnp.int32) for _ in range(NT)]
            + [pltpu.VMEM((L, 2 * D), jnp.float32) for _ in range(NT)]
            + [pltpu.VMEM((L, DOUT), jnp.float32)]
        )

    @functools.partial(
        pl.kernel,
        mesh=mesh,
        out_type=jax.ShapeDtypeStruct((B, L, DOUT), jnp.float32),
        scratch_types=[
            *set_scratch(), *set_scratch(),
            pltpu.SemaphoreType.DMA, pltpu.SemaphoreType.DMA,
            pltpu.SemaphoreType.DMA, pltpu.SemaphoreType.DMA,
            pltpu.SemaphoreType.DMA, pltpu.SemaphoreType.DMA,
        ],
        compiler_params=pltpu.CompilerParams(needs_layout_passes=False,
                                             disable_bounds_checks=True),
    )
    def k(gids, sids, cids, pids, gt2, st2, ct2, pt2, out, *refs):
        wid = lax.axis_index("s") * NC + lax.axis_index("c")
        b0 = wid * B_PER_W

        ids = (gids, sids, cids, pids)
        tabs = (gt2, st2, ct2, pt2)
        per_set = 3 * NT + NT + 1
        sets = []
        for s in range(2):
            r = refs[s * per_set:(s + 1) * per_set]
            sets.append(dict(raw=r[0:NT], gix=r[NT:2 * NT],
                             par=r[2 * NT:3 * NT], bg=r[3 * NT:4 * NT],
                             cc=r[4 * NT]))
        sem_p = refs[2 * per_set:2 * per_set + 2]
        sem_g = refs[2 * per_set + 2:2 * per_set + 4]
        sem_w = refs[2 * per_set + 4:2 * per_set + 6]

        def prefetch_ids(s, batch):
            @pl.when(batch < B_PER_W)
            def _():
                for t in range(NT):
                    pltpu.async_copy(ids[t].at[pl.ds((b0 + batch) * LP, LP)],
                                     sets[s]["raw"][t], sem_p[s])

        def wait_ids(s):
            for t in range(NT):
                pltpu.make_async_copy(ids[t].at[pl.ds(b0 * LP, LP)],
                                      sets[s]["raw"][t], sem_p[s]).wait()

        def build_gix(s):
            st = sets[s]
            for t in range(NT):
                raw, gix, par = st["raw"][t], st["gix"][t], st["par"][t]
                for g in range(4):
                    v = raw[pl.ds(16 * g, 16)]
                    gix[pl.ds(16 * g, 16)] = v >> 1
                    par[pl.ds(16 * g, 16)] = v & 1

        def fire_gathers(s):
            st = sets[s]
            return [pltpu.async_copy(
                        tabs[t].at[st["gix"][t].at[pl.ds(0, L)]],
                        st["bg"][t], sem_g[s])
                    for t in range(NT)]

        iota16 = lax.broadcasted_iota(jnp.int32, (16,), 0)

        cbase = [iota16 + 16 * j for j in range(D // 16)]

        def select(s):
            st = sets[s]
            cc = st["cc"]

            def tok(i, _):
                ri = iota16 * 0 + i
                offs = [plsc.load_gather(st["par"][t], [ri]) * D
                        for t in range(NT)]
                for j in range(D // 16):
                    vals = [plsc.load_gather(st["bg"][t],
                                             [ri, offs[t] + cbase[j]])
                            for t in range(NT)]
                    for t in range(NT):
                        plsc.store_scatter(cc, [ri, cbase[j] + t * D],
                                           vals[t])
                return 0

            lax.fori_loop(0, L, tok, 0)

        def write(s, batch):
            pltpu.async_copy(sets[s]["cc"], out.at[b0 + batch], sem_w[s])

        def drain_write(s):
            pltpu.make_async_copy(out.at[b0], sets[s]["cc"], sem_w[s]).wait()

        prefetch_ids(0, 0)
        prefetch_ids(1, 1)

        def pair(j, _):
            c0, c1 = 2 * j, 2 * j + 1
            wait_ids(0)
            build_gix(0)
            prefetch_ids(0, c0 + 2)
            g0 = fire_gathers(0)

            wait_ids(1)
            build_gix(1)
            prefetch_ids(1, c1 + 2)
            g1 = fire_gathers(1)

            for c in g0:
                c.wait()

            @pl.when(j > 0)
            def _():
                drain_write(0)
            select(0)
            write(0, c0)
            for c in g1:
                c.wait()

            @pl.when(j > 0)
            def _():
                drain_write(1)
            select(1)
            write(1, c1)
            return 0

        lax.fori_loop(0, PAIRS, pair, 0)
        drain_write(0)
        drain_write(1)

    return k


_kern = _make_kernel()


def _pad_ids(x):
    return jnp.pad(x, ((0, 0), (0, 64 - L))).reshape(-1)


def kernel(goods_ids, shop_ids, cate_ids, goods_prices,
           goods_table, shop_table, cate_table, price_table):
    return _kern(_pad_ids(goods_ids), _pad_ids(shop_ids),
                 _pad_ids(cate_ids), _pad_ids(goods_prices),
                 goods_table.reshape(-1, 2 * D),
                 shop_table.reshape(-1, 2 * D),
                 cate_table.reshape(-1, 2 * D),
                 price_table.reshape(-1, 2 * D))
